# baseline (speedup 1.0000x reference)
import functools

import jax
import jax.numpy as jnp
from jax import lax
from jax.experimental import pallas as pl
from jax.experimental.pallas import tpu as pltpu
from jax.experimental.pallas import tpu_sc as plsc

N = 10000
D = 128
E = 320000
NC = 2
NS = 16
NW = NC * NS
CH = 128
CPT = 80
EP = NW * CPT * CH
NP = 10240
RPT = NP // NS


def _sc_mesh():
    return plsc.VectorSubcoreMesh(core_axis_name="c", subcore_axis_name="s")


def _sc_hist(dstp, zflat, ones):
    @functools.partial(
        pl.kernel,
        mesh=_sc_mesh(),
        out_type=jax.ShapeDtypeStruct((NC, NP), jnp.float32),
        scratch_types=[
            pltpu.VMEM_SHARED((NP,), jnp.float32),
            pltpu.VMEM((CPT, CH), jnp.int32),
            pltpu.VMEM((CH,), jnp.float32),
            pltpu.SemaphoreType.DMA,
        ],
    )
    def k(dst_h, zflat_h, ones_h, out_h, hsh, dv, ov, hsem):
        c = lax.axis_index("c")
        s = lax.axis_index("s")
        wid = c * NS + s
        rs = s * RPT
        pltpu.sync_copy(zflat_h, hsh.at[pl.ds(rs, RPT)])
        pltpu.sync_copy(ones_h, ov)
        pltpu.sync_copy(dst_h.at[wid], dv)
        plsc.subcore_barrier()

        W8 = 16

        def body(w, carry):
            for i in range(W8):
                pltpu.async_copy(ov, hsh.at[dv.at[w * W8 + i]], hsem, add=True)
            for i in range(W8):
                pltpu.make_async_copy(ov, hsh.at[dv.at[w * W8 + i]], hsem).wait()
            return carry

        lax.fori_loop(0, CPT // W8, body, 0)
        plsc.subcore_barrier()
        pltpu.sync_copy(hsh.at[pl.ds(rs, RPT)], out_h.at[c, pl.ds(rs, RPT)])

    return k(dstp, zflat, ones)


def _sc_agg(table, srcp, dstp):
    @functools.partial(
        pl.kernel,
        mesh=_sc_mesh(),
        out_type=jax.ShapeDtypeStruct((NC, NP, D), jnp.float32),
        scratch_types=[
            pltpu.VMEM_SHARED((NP, D), jnp.float32),
            pltpu.VMEM((CPT // 2, CH), jnp.int32),
            pltpu.VMEM((CPT // 2, CH), jnp.int32),
            pltpu.VMEM((CH, D), jnp.float32),
            pltpu.VMEM((CH, D), jnp.float32),
            pltpu.SemaphoreType.DMA,
            pltpu.SemaphoreType.DMA,
        ],
    )
    def k(table_h, src_h, dst_h, part_h, acc, sv, dvx, rows0, rows1,
          sem0, sem1):
        c = lax.axis_index("c")
        s = lax.axis_index("s")
        wid = c * NS + s
        rs = s * RPT
        H = CPT // 2

        pltpu.async_copy(src_h.at[wid, pl.ds(0, H)], sv, sem0)
        pltpu.async_copy(dst_h.at[wid, pl.ds(0, H)], dvx, sem1)

        @pl.when(c == 0)
        def _():
            pltpu.sync_copy(table_h.at[pl.ds(rs, RPT)], acc.at[pl.ds(rs, RPT)])

        @pl.when(c != 0)
        def _():
            def zbody(i, carry):
                rows0[i // 8, pl.ds((i % 8) * 16, 16)] = jnp.zeros(
                    (16,), jnp.float32)
                return carry

            lax.fori_loop(0, CH * D // 16, zbody, 0)
            for b in range(RPT // CH):
                pltpu.sync_copy(rows0, acc.at[pl.ds(rs + b * CH, CH)])

        pltpu.make_async_copy(src_h.at[wid, pl.ds(0, H)], sv, sem0).wait()
        pltpu.make_async_copy(dst_h.at[wid, pl.ds(0, H)], dvx, sem1).wait()
        plsc.subcore_barrier()

        for h in range(2):
            if h == 1:
                pltpu.sync_copy(src_h.at[wid, pl.ds(h * H, H)], sv)
                pltpu.sync_copy(dst_h.at[wid, pl.ds(h * H, H)], dvx)
            pltpu.async_copy(table_h.at[sv.at[0]], rows0, sem0)

            def body(k, carry):
                j0 = 2 * k
                j1 = j0 + 1
                pltpu.async_copy(table_h.at[sv.at[j1]], rows1, sem1)
                pltpu.make_async_copy(table_h.at[sv.at[j0]], rows0, sem0).wait()
                pltpu.sync_copy(rows0, acc.at[dvx.at[j0]], add=True)

                @pl.when(k < H // 2 - 1)
                def _():
                    pltpu.async_copy(table_h.at[sv.at[j0 + 2]], rows0, sem0)

                pltpu.make_async_copy(table_h.at[sv.at[j1]], rows1, sem1).wait()
                pltpu.sync_copy(rows1, acc.at[dvx.at[j1]], add=True)
                return carry

            lax.fori_loop(0, H // 2, body, 0)
        plsc.subcore_barrier()
        pltpu.sync_copy(acc.at[pl.ds(rs, RPT)], part_h.at[c, pl.ds(rs, RPT)])

    return k(table, srcp, dstp)


def _dinv_block(hist_blk):
    deg = hist_blk[0, :] + hist_blk[1, :] + 1.0
    return lax.rsqrt(deg)


def _tc1_body(hist_ref, x_ref, w_ref, o_ref):
    dinv = _dinv_block(hist_ref[...])
    xw = jnp.dot(x_ref[...], w_ref[...], preferred_element_type=jnp.float32)
    o_ref[...] = xw * dinv[:, None]


def _tc1(hist, xp, W1):
    return pl.pallas_call(
        _tc1_body,
        out_shape=jax.ShapeDtypeStruct((NP, D), jnp.float32),
    )(hist, xp, W1)


def _tc2_body(p_ref, hist_ref, b_ref, w_ref, o_ref):
    dinv = _dinv_block(hist_ref[...])
    ssum = p_ref[0] + p_ref[1]
    hid = jnp.maximum(ssum * dinv[:, None] + b_ref[...], 0.0)
    hw = jnp.dot(hid, w_ref[...], preferred_element_type=jnp.float32)
    o_ref[...] = hw * dinv[:, None]


def _tc2(part, hist, b1r, W2):
    return pl.pallas_call(
        _tc2_body,
        out_shape=jax.ShapeDtypeStruct((NP, D), jnp.float32),
    )(part, hist, b1r, W2)


def _tc3_body(p_ref, hist_ref, b_ref, o_ref):
    dinv = _dinv_block(hist_ref[...])
    o = (p_ref[0] + p_ref[1]) * dinv[:, None] + b_ref[...]
    m = jnp.max(o, axis=1, keepdims=True)
    z = o - m
    lse = jnp.log(jnp.sum(jnp.exp(z), axis=1, keepdims=True))
    o_ref[...] = z - lse


def _tc3(part, hist, b2r):
    return pl.pallas_call(
        _tc3_body,
        out_shape=jax.ShapeDtypeStruct((NP, D), jnp.float32),
    )(part, hist, b2r)


def kernel(x, edge_index, W1, b1, W2, b2):
    src = edge_index[0]
    dst = edge_index[1]
    pad = N + (jnp.arange(EP - E, dtype=jnp.int32) % (NP - N))
    srcp = jnp.concatenate([src, pad]).reshape(NW, CPT, CH)
    dstp = jnp.concatenate([dst, pad]).reshape(NW, CPT, CH)
    xp = jnp.pad(x, ((0, NP - N), (0, 0)))
    zflat = jnp.zeros((RPT,), jnp.float32)
    ones = jnp.ones((CH,), jnp.float32)

    hist = _sc_hist(dstp, zflat, ones)
    hp1 = _tc1(hist, xp, W1)
    p1 = _sc_agg(hp1, srcp, dstp)
    hp2 = _tc2(p1, hist, b1.reshape(1, D), W2)
    p2 = _sc_agg(hp2, srcp, dstp)
    out = _tc3(p2, hist, b2.reshape(1, D))
    return out[:N]


# speedup vs baseline: 1.0016x; 1.0016x over previous
import functools

import jax
import jax.numpy as jnp
from jax import lax
from jax.experimental import pallas as pl
from jax.experimental.pallas import tpu as pltpu
from jax.experimental.pallas import tpu_sc as plsc

N = 10000
D = 128
E = 320000
NC = 2
NS = 16
NW = NC * NS
CH = 128
CPT = 80
EP = NW * CPT * CH
NP = 10240
RPT = NP // NS


def _sc_mesh():
    return plsc.VectorSubcoreMesh(core_axis_name="c", subcore_axis_name="s")


def _sc_hist(dstp, zflat, ones):
    @functools.partial(
        pl.kernel,
        mesh=_sc_mesh(),
        out_type=jax.ShapeDtypeStruct((NC, NP), jnp.float32),
        scratch_types=[
            pltpu.VMEM_SHARED((NP,), jnp.float32),
            pltpu.VMEM((CPT, CH), jnp.int32),
            pltpu.VMEM((CH,), jnp.float32),
            pltpu.SemaphoreType.DMA,
        ],
    )
    def k(dst_h, zflat_h, ones_h, out_h, hsh, dv, ov, hsem):
        c = lax.axis_index("c")
        s = lax.axis_index("s")
        wid = c * NS + s
        rs = s * RPT
        pltpu.sync_copy(zflat_h, hsh.at[pl.ds(rs, RPT)])
        pltpu.sync_copy(ones_h, ov)
        pltpu.sync_copy(dst_h.at[wid], dv)
        plsc.subcore_barrier()

        W8 = 20

        def body(w, carry):
            for i in range(W8):
                pltpu.async_copy(ov, hsh.at[dv.at[w * W8 + i]], hsem, add=True)
            for i in range(W8):
                pltpu.make_async_copy(ov, hsh.at[dv.at[w * W8 + i]], hsem).wait()
            return carry

        lax.fori_loop(0, CPT // W8, body, 0)
        plsc.subcore_barrier()
        pltpu.sync_copy(hsh.at[pl.ds(rs, RPT)], out_h.at[c, pl.ds(rs, RPT)])

    return k(dstp, zflat, ones)


def _sc_agg(table, srcp, dstp):
    @functools.partial(
        pl.kernel,
        mesh=_sc_mesh(),
        out_type=jax.ShapeDtypeStruct((NC, NP, D), jnp.float32),
        scratch_types=[
            pltpu.VMEM_SHARED((NP, D), jnp.float32),
            pltpu.VMEM((CPT // 2, CH), jnp.int32),
            pltpu.VMEM((CPT // 2, CH), jnp.int32),
            pltpu.VMEM((CH, D), jnp.float32),
            pltpu.VMEM((CH, D), jnp.float32),
            pltpu.SemaphoreType.DMA,
            pltpu.SemaphoreType.DMA,
        ],
    )
    def k(table_h, src_h, dst_h, part_h, acc, sv, dvx, rows0, rows1,
          sem0, sem1):
        c = lax.axis_index("c")
        s = lax.axis_index("s")
        wid = c * NS + s
        rs = s * RPT
        H = CPT // 2

        pltpu.async_copy(src_h.at[wid, pl.ds(0, H)], sv, sem0)
        pltpu.async_copy(dst_h.at[wid, pl.ds(0, H)], dvx, sem1)

        @pl.when(c == 0)
        def _():
            pltpu.sync_copy(table_h.at[pl.ds(rs, RPT)], acc.at[pl.ds(rs, RPT)])

        @pl.when(c != 0)
        def _():
            def zbody(i, carry):
                rows0[i // 8, pl.ds((i % 8) * 16, 16)] = jnp.zeros(
                    (16,), jnp.float32)
                return carry

            lax.fori_loop(0, CH * D // 16, zbody, 0)
            for b in range(RPT // CH):
                pltpu.sync_copy(rows0, acc.at[pl.ds(rs + b * CH, CH)])

        pltpu.make_async_copy(src_h.at[wid, pl.ds(0, H)], sv, sem0).wait()
        pltpu.make_async_copy(dst_h.at[wid, pl.ds(0, H)], dvx, sem1).wait()
        plsc.subcore_barrier()

        for h in range(2):
            if h == 1:
                pltpu.sync_copy(src_h.at[wid, pl.ds(h * H, H)], sv)
                pltpu.sync_copy(dst_h.at[wid, pl.ds(h * H, H)], dvx)
            pltpu.async_copy(table_h.at[sv.at[0]], rows0, sem0)

            def body(k, carry):
                j0 = 2 * k
                j1 = j0 + 1
                pltpu.async_copy(table_h.at[sv.at[j1]], rows1, sem1)
                pltpu.make_async_copy(table_h.at[sv.at[j0]], rows0, sem0).wait()
                pltpu.sync_copy(rows0, acc.at[dvx.at[j0]], add=True)

                @pl.when(k < H // 2 - 1)
                def _():
                    pltpu.async_copy(table_h.at[sv.at[j0 + 2]], rows0, sem0)

                pltpu.make_async_copy(table_h.at[sv.at[j1]], rows1, sem1).wait()
                pltpu.sync_copy(rows1, acc.at[dvx.at[j1]], add=True)
                return carry

            lax.fori_loop(0, H // 2, body, 0)
        plsc.subcore_barrier()
        pltpu.sync_copy(acc.at[pl.ds(rs, RPT)], part_h.at[c, pl.ds(rs, RPT)])

    return k(table, srcp, dstp)


def _dinv_block(hist_blk):
    deg = hist_blk[0, :] + hist_blk[1, :] + 1.0
    return lax.rsqrt(deg)


def _tc1_body(hist_ref, x_ref, w_ref, o_ref):
    dinv = _dinv_block(hist_ref[...])
    xw = jnp.dot(x_ref[...], w_ref[...], preferred_element_type=jnp.float32)
    o_ref[...] = xw * dinv[:, None]


def _tc1(hist, xp, W1):
    return pl.pallas_call(
        _tc1_body,
        out_shape=jax.ShapeDtypeStruct((NP, D), jnp.float32),
    )(hist, xp, W1)


def _tc2_body(p_ref, hist_ref, b_ref, w_ref, o_ref):
    dinv = _dinv_block(hist_ref[...])
    ssum = p_ref[0] + p_ref[1]
    hid = jnp.maximum(ssum * dinv[:, None] + b_ref[...], 0.0)
    hw = jnp.dot(hid, w_ref[...], preferred_element_type=jnp.float32)
    o_ref[...] = hw * dinv[:, None]


def _tc2(part, hist, b1r, W2):
    return pl.pallas_call(
        _tc2_body,
        out_shape=jax.ShapeDtypeStruct((NP, D), jnp.float32),
    )(part, hist, b1r, W2)


def _tc3_body(p_ref, hist_ref, b_ref, o_ref):
    dinv = _dinv_block(hist_ref[...])
    o = (p_ref[0] + p_ref[1]) * dinv[:, None] + b_ref[...]
    m = jnp.max(o, axis=1, keepdims=True)
    z = o - m
    lse = jnp.log(jnp.sum(jnp.exp(z), axis=1, keepdims=True))
    o_ref[...] = z - lse


def _tc3(part, hist, b2r):
    return pl.pallas_call(
        _tc3_body,
        out_shape=jax.ShapeDtypeStruct((NP, D), jnp.float32),
    )(part, hist, b2r)


def kernel(x, edge_index, W1, b1, W2, b2):
    src = edge_index[0]
    dst = edge_index[1]
    pad = N + (jnp.arange(EP - E, dtype=jnp.int32) % (NP - N))
    srcp = jnp.concatenate([src, pad]).reshape(NW, CPT, CH)
    dstp = jnp.concatenate([dst, pad]).reshape(NW, CPT, CH)
    xp = jnp.pad(x, ((0, NP - N), (0, 0)))
    zflat = jnp.zeros((RPT,), jnp.float32)
    ones = jnp.ones((CH,), jnp.float32)

    hist = _sc_hist(dstp, zflat, ones)
    hp1 = _tc1(hist, xp, W1)
    p1 = _sc_agg(hp1, srcp, dstp)
    hp2 = _tc2(p1, hist, b1.reshape(1, D), W2)
    p2 = _sc_agg(hp2, srcp, dstp)
    out = _tc3(p2, hist, b2.reshape(1, D))
    return out[:N]
